# Initial kernel scaffold; baseline (speedup 1.0000x reference)
import functools

import jax
import jax.numpy as jnp
from jax import lax
from jax.experimental import pallas as pl
from jax.experimental.pallas import tpu as pltpu
from jax.experimental.pallas import tpu_sc as plsc

N_POI = 10000
E = 160000
D = 128
B = 1024
N_ITEMS = 20480
NPAD = 10240

_MESH = dict(core_axis_name="c", subcore_axis_name="s")


def _sc_deg(edges_flat):
    mesh = plsc.VectorSubcoreMesh(**_MESH)
    per_w = (2 * E) // 32
    nfull = per_w // 128
    tail = per_w - nfull * 128

    @functools.partial(
        pl.kernel, mesh=mesh,
        out_type=jax.ShapeDtypeStruct((2, NPAD), jnp.float32),
        scratch_types=[
            pltpu.VMEM((128,), jnp.int32),
            pltpu.VMEM((tail,), jnp.int32),
            pltpu.VMEM((128,), jnp.float32),
            pltpu.VMEM_SHARED((NPAD,), jnp.float32),
        ],
    )
    def k(e_hbm, out_hbm, idx_v, idx_t, ones_v, acc):
        cid = lax.axis_index("c")
        sid = lax.axis_index("s")
        w = cid * 16 + sid
        onev = jnp.ones((16,), jnp.float32)
        zv = jnp.zeros((16,), jnp.float32)
        for j in range(8):
            ones_v[pl.ds(j * 16, 16)] = zv

        @pl.loop(0, 5)
        def _(i):
            pltpu.sync_copy(ones_v, acc.at[pl.ds(sid * 640 + i * 128, 128)])

        for j in range(8):
            ones_v[pl.ds(j * 16, 16)] = onev
        plsc.subcore_barrier()

        base0 = w * per_w

        @pl.loop(0, nfull)
        def _(i):
            pltpu.sync_copy(e_hbm.at[pl.ds(base0 + i * 128, 128)], idx_v)
            pltpu.sync_copy(ones_v, acc.at[idx_v], add=True)

        pltpu.sync_copy(e_hbm.at[pl.ds(base0 + nfull * 128, tail)], idx_t)
        pltpu.sync_copy(ones_v.at[pl.ds(0, tail)], acc.at[idx_t], add=True)
        plsc.subcore_barrier()
        pltpu.sync_copy(acc.at[pl.ds(sid * 640, 640)],
                        out_hbm.at[cid, pl.ds(sid * 640, 640)])

    return k(edges_flat)


def _sc_aggr(z, e0, e1, w):
    mesh = plsc.VectorSubcoreMesh(**_MESH)
    per_w = E // 32
    nfull = per_w // 128
    tail = per_w - nfull * 128

    @functools.partial(
        pl.kernel, mesh=mesh,
        out_type=jax.ShapeDtypeStruct((2, NPAD, D), jnp.float32),
        scratch_types=[
            pltpu.VMEM((128,), jnp.int32),
            pltpu.VMEM((128,), jnp.int32),
            pltpu.VMEM((tail,), jnp.int32),
            pltpu.VMEM((tail,), jnp.int32),
            pltpu.VMEM((128,), jnp.float32),
            pltpu.VMEM((128, D), jnp.float32),
            pltpu.VMEM((128, D), jnp.float32),
            pltpu.VMEM_SHARED((NPAD, D), jnp.float32),
        ],
    )
    def k(z_hbm, e0_hbm, e1_hbm, w_hbm, out_hbm,
          iA, iB, tA, tB, wv, buf, zb, acc):
        cid = lax.axis_index("c")
        sid = lax.axis_index("s")
        wkr = cid * 16 + sid
        zv = jnp.zeros((16,), jnp.float32)

        @pl.loop(0, 128)
        def _(r):
            for j in range(8):
                zb[r, pl.ds(j * 16, 16)] = zv

        @pl.loop(0, 5)
        def _(i):
            pltpu.sync_copy(zb, acc.at[pl.ds(sid * 640 + i * 128, 128)])

        plsc.subcore_barrier()
        base0 = wkr * per_w

        def scale(n):
            @pl.loop(0, n)
            def _(r):
                ridx = lax.broadcast_in_dim(r, (16,), ())
                ws = plsc.load_gather(wv, [ridx])
                for j in range(8):
                    buf[r, pl.ds(j * 16, 16)] = buf[r, pl.ds(j * 16, 16)] * ws

        def chunk(base, n, ia, ib):
            pltpu.sync_copy(e0_hbm.at[pl.ds(base, n)], ia)
            pltpu.sync_copy(e1_hbm.at[pl.ds(base, n)], ib)
            pltpu.sync_copy(w_hbm.at[pl.ds(base, n)], wv.at[pl.ds(0, n)])
            dst = buf if n == 128 else buf.at[pl.ds(0, n)]
            pltpu.sync_copy(z_hbm.at[ib], dst)
            scale(n)
            pltpu.sync_copy(dst, acc.at[ia], add=True)
            pltpu.sync_copy(z_hbm.at[ia], dst)
            scale(n)
            pltpu.sync_copy(dst, acc.at[ib], add=True)

        @pl.loop(0, nfull)
        def _(i):
            chunk(base0 + i * 128, 128, iA, iB)

        chunk(base0 + nfull * 128, tail, tA, tB)
        plsc.subcore_barrier()
        pltpu.sync_copy(acc.at[pl.ds(sid * 640, 640)],
                        out_hbm.at[cid].at[pl.ds(sid * 640, 640)])

    return k(z, e0, e1, w)


def _sc_gather(enc, x, poi):
    mesh = plsc.VectorSubcoreMesh(**_MESH)

    @functools.partial(
        pl.kernel, mesh=mesh,
        out_type=[jax.ShapeDtypeStruct((N_ITEMS, D), jnp.float32),
                  jax.ShapeDtypeStruct((B, D), jnp.float32)],
        scratch_types=[
            pltpu.VMEM((128,), jnp.int32),
            pltpu.VMEM((32,), jnp.int32),
            pltpu.VMEM((128, D), jnp.float32),
        ],
    )
    def k(enc_hbm, x_hbm, poi_hbm, geo_hbm, tar_hbm, ix, ip, buf):
        cid = lax.axis_index("c")
        sid = lax.axis_index("s")
        wkr = cid * 16 + sid

        @pl.loop(0, N_ITEMS // (32 * 128))
        def _(i):
            base = wkr * (N_ITEMS // 32) + i * 128
            pltpu.sync_copy(x_hbm.at[pl.ds(base, 128)], ix)
            pltpu.sync_copy(enc_hbm.at[ix], buf)
            pltpu.sync_copy(buf, geo_hbm.at[pl.ds(base, 128)])

        base = wkr * (B // 32)
        pltpu.sync_copy(poi_hbm.at[pl.ds(base, 32)], ip)
        pltpu.sync_copy(enc_hbm.at[ip], buf.at[pl.ds(0, 32)])
        pltpu.sync_copy(buf.at[pl.ds(0, 32)], tar_hbm.at[pl.ds(base, 32)])

    return k(enc, x, poi)


def _tc_prep_body(dv_ref, pt_ref, degT_ref, w_ref, z_ref):
    w_ref[...] = jnp.exp(-(dv_ref[...] ** 2))
    deg = degT_ref[...]
    dis = lax.rsqrt(deg[:, 0:1] + deg[:, 1:2] + 1.0)
    z_ref[...] = pt_ref[...] * dis


def _tc_prep(dv2d, pt_pad, degT):
    return pl.pallas_call(
        _tc_prep_body,
        out_shape=[jax.ShapeDtypeStruct(dv2d.shape, jnp.float32),
                   jax.ShapeDtypeStruct((NPAD, D), jnp.float32)],
    )(dv2d, pt_pad, degT)


def _tc_layer_body(u_ref, z_ref, degT_ref, w_ref, b_ref, enc_ref, zn_ref):
    deg = degT_ref[...]
    dis = lax.rsqrt(deg[:, 0:1] + deg[:, 1:2] + 1.0)
    side = dis * (u_ref[0] + u_ref[1] + z_ref[...])
    h = lax.dot_general(side, w_ref[...], (((1,), (1,)), ((), ())),
                        preferred_element_type=jnp.float32) + b_ref[...]
    h = jnp.where(h >= 0, h, 0.01 * h)
    nrm = jnp.sqrt(jnp.sum(h * h, axis=1, keepdims=True))
    enc = h / jnp.maximum(nrm, 1e-12)
    enc_ref[...] = enc
    zn_ref[...] = enc * dis


def _tc_layer(u, z, degT, w, b2d):
    return pl.pallas_call(
        _tc_layer_body,
        out_shape=[jax.ShapeDtypeStruct((NPAD, D), jnp.float32),
                   jax.ShapeDtypeStruct((NPAD, D), jnp.float32)],
    )(u, z, degT, w, b2d)


_BLK = 2048
_NBLK = N_ITEMS // _BLK


def _dot_t(a, w_ref):
    return lax.dot_general(a, w_ref[...], (((1,), (1,)), ((), ())),
                           preferred_element_type=jnp.float32)


def _lrelu(h):
    return jnp.where(h >= 0, h, 0.01 * h)


def _tc_attn_body(geo_ref, b3_ref, tar_ref, Kw, Kb, Qw, Qb, Vw, Vb,
                  P1w, P1b, P2w, P2b, J1w, J1b, J2w, J2b,
                  out1_ref, pred_ref, q_s, accM, acc2):
    i = pl.program_id(0)

    @pl.when(i == 0)
    def _():
        q_s[...] = _dot_t(tar_ref[...], Qw) + Qb[...]
        accM[...] = jnp.zeros_like(accM)
        acc2[...] = jnp.zeros_like(acc2)

    g = geo_ref[...]
    v = _dot_t(g, Kw) + Kb[...]
    bcol = b3_ref[0]
    P = (bcol == lax.broadcasted_iota(jnp.int32, (_BLK, B), 1)
         ).astype(jnp.float32)
    qb = lax.dot_general(P, q_s[...], (((1,), (0,)), ((), ())),
                         preferred_element_type=jnp.float32)
    logit = jnp.sum(v * qb, axis=1, keepdims=True)
    e = jnp.exp(logit)
    M = jnp.concatenate([v * e, g], axis=1)
    accM[...] += lax.dot_general(P, M, (((0,), (0,)), ((), ())),
                                 preferred_element_type=jnp.float32)
    ones = jnp.ones((_BLK, 1), jnp.float32)
    M2 = jnp.concatenate([e, ones, jnp.zeros((_BLK, 6), jnp.float32)], axis=1)
    acc2[...] += lax.dot_general(P, M2, (((0,), (0,)), ((), ())),
                                 preferred_element_type=jnp.float32)

    @pl.when(i == _NBLK - 1)
    def _():
        den = acc2[:, 0:1]
        cnt = acc2[:, 1:2]
        seq = accM[:, 0:D] / den
        gsum = accM[:, D:2 * D] / cnt
        aggr = _dot_t(seq, Vw) + Vb[...]
        predin = jnp.concatenate([aggr, tar_ref[...]], axis=1)
        h = _lrelu(_dot_t(predin, P1w) + P1b[...])
        pred_ref[...] = _dot_t(h, P2w) + P2b[...]
        gh = _lrelu(_dot_t(gsum, J1w) + J1b[...])
        out1_ref[...] = _dot_t(gh, J2w) + J2b[...]


def _tc_attn(geo, batch3, tar, Kw, Kb, Qw, Qb, Vw, Vb,
             p1w, p1b, p2w, p2b, j1w, j1b, j2w, j2b):
    def full(s):
        return pl.BlockSpec(s, lambda *_: tuple(0 for _ in s))
    return pl.pallas_call(
        _tc_attn_body,
        grid=(_NBLK,),
        in_specs=[
            pl.BlockSpec((_BLK, D), lambda i: (i, 0)),
            pl.BlockSpec((1, _BLK, 1), lambda i: (i, 0, 0)),
            full((B, D)),
            full((D, D)), full((1, D)),
            full((D, D)), full((1, D)),
            full((D, D)), full((1, D)),
            full((D, 2 * D)), full((1, D)),
            full((1, D)), full((1, 1)),
            full((D, D)), full((1, D)),
            full((D, D)), full((1, D)),
        ],
        out_specs=[full((B, D)), full((B, 1))],
        out_shape=[jax.ShapeDtypeStruct((B, D), jnp.float32),
                   jax.ShapeDtypeStruct((B, 1), jnp.float32)],
        scratch_shapes=[pltpu.VMEM((B, D), jnp.float32),
                        pltpu.VMEM((B, 2 * D), jnp.float32),
                        pltpu.VMEM((B, 8), jnp.float32)],
    )(geo, batch3, tar, Kw, Kb, Qw, Qb, Vw, Vb,
      p1w, p1b, p2w, p2b, j1w, j1b, j2w, j2b)


def kernel(dist_edges, dist_vec, batch, poi, x, poi_table,
           gcn0_w, gcn0_b, gcn1_w, gcn1_b, K_w, K_b, Q_w, Q_b, V_w, V_b,
           proj1_w, proj1_b, proj2_w, proj2_b,
           pred1_w, pred1_b, pred2_w, pred2_b):
    dist_edges = dist_edges.astype(jnp.int32)
    e0 = dist_edges[0]
    e1 = dist_edges[1]
    edges_flat = dist_edges.reshape(2 * E)
    batch3 = batch.astype(jnp.int32).reshape(_NBLK, _BLK, 1)
    poi = poi.astype(jnp.int32)
    x = x.astype(jnp.int32)
    pt_pad = jnp.pad(poi_table, ((0, NPAD - N_POI), (0, 0)))
    dv2d = dist_vec.reshape(E // D, D)

    def r1(b):
        return b.reshape(1, -1)

    deg2 = _sc_deg(edges_flat)
    degT = deg2.T
    w2d, z1 = _tc_prep(dv2d, pt_pad, degT)
    w_flat = w2d.reshape(E)
    u1 = _sc_aggr(z1, e0, e1, w_flat)
    _enc1, z2 = _tc_layer(u1, z1, degT, gcn0_w, r1(gcn0_b))
    u2 = _sc_aggr(z2, e0, e1, w_flat)
    enc2, _z3 = _tc_layer(u2, z2, degT, gcn1_w, r1(gcn1_b))
    geo, tar = _sc_gather(enc2, x, poi)
    out1, pred = _tc_attn(
        geo, batch3, tar,
        K_w, r1(K_b), Q_w, r1(Q_b), V_w, r1(V_b),
        pred1_w, r1(pred1_b), pred2_w, pred2_b.reshape(1, 1),
        proj1_w, r1(proj1_b), proj2_w, r1(proj2_b))
    return (out1, pred)


# speedup vs baseline: 12.5280x; 12.5280x over previous
import dataclasses
import functools

import jax
import jax.numpy as jnp
from jax import lax
from jax.experimental import pallas as pl
from jax.experimental.pallas import tpu as pltpu
from jax.experimental.pallas import tpu_sc as plsc

N_POI = 10000
E = 160000
D = 128
B = 1024
N_ITEMS = 20480
NPAD = 10240

_MESH = dict(core_axis_name="c", subcore_axis_name="s")


def _sc_params():
    cp = pltpu.CompilerParams()
    if "needs_layout_passes" in pltpu.CompilerParams.__dataclass_fields__:
        cp = dataclasses.replace(cp, needs_layout_passes=False)
    return cp


def _sc_deg(edges_flat):
    mesh = plsc.VectorSubcoreMesh(**_MESH)
    per_w = (2 * E) // 32
    nfull = per_w // 128
    tail = per_w - nfull * 128

    @functools.partial(
        pl.kernel, mesh=mesh,
        out_type=jax.ShapeDtypeStruct((2, NPAD), jnp.float32),
        scratch_types=[
            pltpu.VMEM((128,), jnp.int32),
            pltpu.VMEM((tail,), jnp.int32),
            pltpu.VMEM((128,), jnp.float32),
            pltpu.VMEM_SHARED((NPAD,), jnp.float32),
        ],
    )
    def k(e_hbm, out_hbm, idx_v, idx_t, ones_v, acc):
        cid = lax.axis_index("c")
        sid = lax.axis_index("s")
        w = cid * 16 + sid
        onev = jnp.ones((16,), jnp.float32)
        zv = jnp.zeros((16,), jnp.float32)
        for j in range(8):
            ones_v[pl.ds(j * 16, 16)] = zv

        @pl.loop(0, 5)
        def _(i):
            pltpu.sync_copy(ones_v, acc.at[pl.ds(sid * 640 + i * 128, 128)])

        for j in range(8):
            ones_v[pl.ds(j * 16, 16)] = onev
        plsc.subcore_barrier()

        base0 = w * per_w

        @pl.loop(0, nfull)
        def _(i):
            pltpu.sync_copy(e_hbm.at[pl.ds(base0 + i * 128, 128)], idx_v)
            pltpu.sync_copy(ones_v, acc.at[idx_v], add=True)

        pltpu.sync_copy(e_hbm.at[pl.ds(base0 + nfull * 128, tail)], idx_t)
        pltpu.sync_copy(ones_v.at[pl.ds(0, tail)], acc.at[idx_t], add=True)
        plsc.subcore_barrier()
        pltpu.sync_copy(acc.at[pl.ds(sid * 640, 640)],
                        out_hbm.at[cid, pl.ds(sid * 640, 640)])

    return k(edges_flat)


def _sc_aggr(z, e0, e1, w):
    mesh = plsc.VectorSubcoreMesh(**_MESH)
    per_w = E // 32
    nfull = per_w // 128
    tail = per_w - nfull * 128

    @functools.partial(
        pl.kernel, mesh=mesh, compiler_params=_sc_params(),
        out_type=jax.ShapeDtypeStruct((2, NPAD, D), jnp.float32),
        scratch_types=[
            pltpu.VMEM((128,), jnp.int32),
            pltpu.VMEM((128,), jnp.int32),
            pltpu.VMEM((tail,), jnp.int32),
            pltpu.VMEM((tail,), jnp.int32),
            pltpu.VMEM((128,), jnp.float32),
            pltpu.VMEM((128, D), jnp.float32),
            pltpu.VMEM((128, D), jnp.float32),
            pltpu.VMEM_SHARED((NPAD, D), jnp.float32),
        ],
    )
    def k(z_hbm, e0_hbm, e1_hbm, w_hbm, out_hbm,
          iA, iB, tA, tB, wv, buf, zb, acc):
        cid = lax.axis_index("c")
        sid = lax.axis_index("s")
        wkr = cid * 16 + sid
        zv = jnp.zeros((16,), jnp.float32)

        @pl.loop(0, 128)
        def _(r):
            for j in range(8):
                zb[r, pl.ds(j * 16, 16)] = zv

        @pl.loop(0, 5)
        def _(i):
            pltpu.sync_copy(zb, acc.at[pl.ds(sid * 640 + i * 128, 128)])

        plsc.subcore_barrier()
        base0 = wkr * per_w

        def scale(n):
            @pl.loop(0, n)
            def _(r):
                ridx = lax.broadcast_in_dim(r, (16,), ())
                ws = plsc.load_gather(wv, [ridx])
                for j in range(8):
                    buf[r, pl.ds(j * 16, 16)] = buf[r, pl.ds(j * 16, 16)] * ws

        def chunk(base, n, ia, ib):
            pltpu.sync_copy(e0_hbm.at[pl.ds(base, n)], ia)
            pltpu.sync_copy(e1_hbm.at[pl.ds(base, n)], ib)
            pltpu.sync_copy(w_hbm.at[pl.ds(base, n)], wv.at[pl.ds(0, n)])
            dst = buf if n == 128 else buf.at[pl.ds(0, n)]
            pltpu.sync_copy(z_hbm.at[ib], dst)
            scale(n)
            pltpu.sync_copy(dst, acc.at[ia], add=True)
            pltpu.sync_copy(z_hbm.at[ia], dst)
            scale(n)
            pltpu.sync_copy(dst, acc.at[ib], add=True)

        @pl.loop(0, nfull)
        def _(i):
            chunk(base0 + i * 128, 128, iA, iB)

        chunk(base0 + nfull * 128, tail, tA, tB)
        plsc.subcore_barrier()
        pltpu.sync_copy(acc.at[pl.ds(sid * 640, 640)],
                        out_hbm.at[cid].at[pl.ds(sid * 640, 640)])

    return k(z, e0, e1, w)


def _sc_gather(enc, x, poi):
    mesh = plsc.VectorSubcoreMesh(**_MESH)

    @functools.partial(
        pl.kernel, mesh=mesh,
        out_type=[jax.ShapeDtypeStruct((N_ITEMS, D), jnp.float32),
                  jax.ShapeDtypeStruct((B, D), jnp.float32)],
        scratch_types=[
            pltpu.VMEM((128,), jnp.int32),
            pltpu.VMEM((32,), jnp.int32),
            pltpu.VMEM((128, D), jnp.float32),
        ],
    )
    def k(enc_hbm, x_hbm, poi_hbm, geo_hbm, tar_hbm, ix, ip, buf):
        cid = lax.axis_index("c")
        sid = lax.axis_index("s")
        wkr = cid * 16 + sid

        @pl.loop(0, N_ITEMS // (32 * 128))
        def _(i):
            base = wkr * (N_ITEMS // 32) + i * 128
            pltpu.sync_copy(x_hbm.at[pl.ds(base, 128)], ix)
            pltpu.sync_copy(enc_hbm.at[ix], buf)
            pltpu.sync_copy(buf, geo_hbm.at[pl.ds(base, 128)])

        base = wkr * (B // 32)
        pltpu.sync_copy(poi_hbm.at[pl.ds(base, 32)], ip)
        pltpu.sync_copy(enc_hbm.at[ip], buf.at[pl.ds(0, 32)])
        pltpu.sync_copy(buf.at[pl.ds(0, 32)], tar_hbm.at[pl.ds(base, 32)])

    return k(enc, x, poi)


def _tc_prep_body(dv_ref, pt_ref, degT_ref, w_ref, z_ref):
    w_ref[...] = jnp.exp(-(dv_ref[...] ** 2))
    deg = degT_ref[...]
    dis = lax.rsqrt(deg[:, 0:1] + deg[:, 1:2] + 1.0)
    z_ref[...] = pt_ref[...] * dis


def _tc_prep(dv2d, pt_pad, degT):
    return pl.pallas_call(
        _tc_prep_body,
        out_shape=[jax.ShapeDtypeStruct(dv2d.shape, jnp.float32),
                   jax.ShapeDtypeStruct((NPAD, D), jnp.float32)],
    )(dv2d, pt_pad, degT)


def _tc_layer_body(u_ref, z_ref, degT_ref, w_ref, b_ref, enc_ref, zn_ref):
    deg = degT_ref[...]
    dis = lax.rsqrt(deg[:, 0:1] + deg[:, 1:2] + 1.0)
    side = dis * (u_ref[0] + u_ref[1] + z_ref[...])
    h = lax.dot_general(side, w_ref[...], (((1,), (1,)), ((), ())),
                        preferred_element_type=jnp.float32) + b_ref[...]
    h = jnp.where(h >= 0, h, 0.01 * h)
    nrm = jnp.sqrt(jnp.sum(h * h, axis=1, keepdims=True))
    enc = h / jnp.maximum(nrm, 1e-12)
    enc_ref[...] = enc
    zn_ref[...] = enc * dis


def _tc_layer(u, z, degT, w, b2d):
    return pl.pallas_call(
        _tc_layer_body,
        out_shape=[jax.ShapeDtypeStruct((NPAD, D), jnp.float32),
                   jax.ShapeDtypeStruct((NPAD, D), jnp.float32)],
    )(u, z, degT, w, b2d)


_BLK = 2048
_NBLK = N_ITEMS // _BLK


def _dot_t(a, w_ref):
    return lax.dot_general(a, w_ref[...], (((1,), (1,)), ((), ())),
                           preferred_element_type=jnp.float32)


def _lrelu(h):
    return jnp.where(h >= 0, h, 0.01 * h)


def _tc_attn_body(geo_ref, b3_ref, tar_ref, Kw, Kb, Qw, Qb, Vw, Vb,
                  P1w, P1b, P2w, P2b, J1w, J1b, J2w, J2b,
                  out1_ref, pred_ref, q_s, accM, acc2):
    i = pl.program_id(0)

    @pl.when(i == 0)
    def _():
        q_s[...] = _dot_t(tar_ref[...], Qw) + Qb[...]
        accM[...] = jnp.zeros_like(accM)
        acc2[...] = jnp.zeros_like(acc2)

    g = geo_ref[...]
    v = _dot_t(g, Kw) + Kb[...]
    bcol = b3_ref[0]
    P = (bcol == lax.broadcasted_iota(jnp.int32, (_BLK, B), 1)
         ).astype(jnp.float32)
    qb = lax.dot_general(P, q_s[...], (((1,), (0,)), ((), ())),
                         preferred_element_type=jnp.float32)
    logit = jnp.sum(v * qb, axis=1, keepdims=True)
    e = jnp.exp(logit)
    M = jnp.concatenate([v * e, g], axis=1)
    accM[...] += lax.dot_general(P, M, (((0,), (0,)), ((), ())),
                                 preferred_element_type=jnp.float32)
    ones = jnp.ones((_BLK, 1), jnp.float32)
    M2 = jnp.concatenate([e, ones, jnp.zeros((_BLK, 6), jnp.float32)], axis=1)
    acc2[...] += lax.dot_general(P, M2, (((0,), (0,)), ((), ())),
                                 preferred_element_type=jnp.float32)

    @pl.when(i == _NBLK - 1)
    def _():
        den = acc2[:, 0:1]
        cnt = acc2[:, 1:2]
        seq = accM[:, 0:D] / den
        gsum = accM[:, D:2 * D] / cnt
        aggr = _dot_t(seq, Vw) + Vb[...]
        predin = jnp.concatenate([aggr, tar_ref[...]], axis=1)
        h = _lrelu(_dot_t(predin, P1w) + P1b[...])
        pred_ref[...] = _dot_t(h, P2w) + P2b[...]
        gh = _lrelu(_dot_t(gsum, J1w) + J1b[...])
        out1_ref[...] = _dot_t(gh, J2w) + J2b[...]


def _tc_attn(geo, batch3, tar, Kw, Kb, Qw, Qb, Vw, Vb,
             p1w, p1b, p2w, p2b, j1w, j1b, j2w, j2b):
    def full(s):
        return pl.BlockSpec(s, lambda *_: tuple(0 for _ in s))
    return pl.pallas_call(
        _tc_attn_body,
        grid=(_NBLK,),
        in_specs=[
            pl.BlockSpec((_BLK, D), lambda i: (i, 0)),
            pl.BlockSpec((1, _BLK, 1), lambda i: (i, 0, 0)),
            full((B, D)),
            full((D, D)), full((1, D)),
            full((D, D)), full((1, D)),
            full((D, D)), full((1, D)),
            full((D, 2 * D)), full((1, D)),
            full((8, D)), full((1, 8)),
            full((D, D)), full((1, D)),
            full((D, D)), full((1, D)),
        ],
        out_specs=[full((B, D)), full((B, 8))],
        out_shape=[jax.ShapeDtypeStruct((B, D), jnp.float32),
                   jax.ShapeDtypeStruct((B, 8), jnp.float32)],
        scratch_shapes=[pltpu.VMEM((B, D), jnp.float32),
                        pltpu.VMEM((B, 2 * D), jnp.float32),
                        pltpu.VMEM((B, 8), jnp.float32)],
    )(geo, batch3, tar, Kw, Kb, Qw, Qb, Vw, Vb,
      p1w, p1b, p2w, p2b, j1w, j1b, j2w, j2b)


def kernel(dist_edges, dist_vec, batch, poi, x, poi_table,
           gcn0_w, gcn0_b, gcn1_w, gcn1_b, K_w, K_b, Q_w, Q_b, V_w, V_b,
           proj1_w, proj1_b, proj2_w, proj2_b,
           pred1_w, pred1_b, pred2_w, pred2_b):
    dist_edges = dist_edges.astype(jnp.int32)
    e0 = dist_edges[0]
    e1 = dist_edges[1]
    edges_flat = dist_edges.reshape(2 * E)
    batch3 = batch.astype(jnp.int32).reshape(_NBLK, _BLK, 1)
    poi = poi.astype(jnp.int32)
    x = x.astype(jnp.int32)
    pt_pad = jnp.pad(poi_table, ((0, NPAD - N_POI), (0, 0)))
    dv2d = dist_vec.reshape(E // D, D)

    def r1(b):
        return b.reshape(1, -1)

    deg2 = _sc_deg(edges_flat)
    degT = deg2.T
    w2d, z1 = _tc_prep(dv2d, pt_pad, degT)
    w_flat = w2d.reshape(E)
    u1 = _sc_aggr(z1, e0, e1, w_flat)
    _enc1, z2 = _tc_layer(u1, z1, degT, gcn0_w, r1(gcn0_b))
    u2 = _sc_aggr(z2, e0, e1, w_flat)
    enc2, _z3 = _tc_layer(u2, z2, degT, gcn1_w, r1(gcn1_b))
    geo, tar = _sc_gather(enc2, x, poi)
    p2w8 = jnp.pad(pred2_w, ((0, 7), (0, 0)))
    p2b8 = jnp.pad(pred2_b.reshape(1, 1), ((0, 0), (0, 7)))
    out1, pred8 = _tc_attn(
        geo, batch3, tar,
        K_w, r1(K_b), Q_w, r1(Q_b), V_w, r1(V_b),
        pred1_w, r1(pred1_b), p2w8, p2b8,
        proj1_w, r1(proj1_b), proj2_w, r1(proj2_b))
    return (out1, pred8[:, 0:1])
